# Initial kernel scaffold; baseline (speedup 1.0000x reference)
import functools

import jax
import jax.numpy as jnp
from jax import lax
from jax.experimental import pallas as pl
from jax.experimental.pallas import tpu as pltpu
from jax.experimental.pallas import tpu_sc as plsc

N = 10000
E = 320000
E2 = E + N
NW = 32
NPT = 320
NPAD = NW * NPT
EPT = ((E2 + NW * 16 - 1) // (NW * 16)) * 16
E2P = EPT * NW
K = 32
RPLEN = NPT + 8


def _wid():
    return lax.axis_index("s") * 2 + lax.axis_index("c")


def _mm(h, w, b, relu):
    m, kd = h.shape
    nd = w.shape[1]
    bm = 512

    def body(h_ref, w_ref, b_ref, o_ref):
        acc = jnp.dot(h_ref[...], w_ref[...], preferred_element_type=jnp.float32)
        acc = acc + b_ref[...]
        if relu:
            acc = jnp.maximum(acc, 0.0)
        o_ref[...] = acc

    return pl.pallas_call(
        body,
        grid=(m // bm,),
        in_specs=[
            pl.BlockSpec((bm, kd), lambda i: (i, 0)),
            pl.BlockSpec((kd, nd), lambda i: (0, 0)),
            pl.BlockSpec((1, nd), lambda i: (0, 0)),
        ],
        out_specs=pl.BlockSpec((bm, nd), lambda i: (i, 0)),
        out_shape=jax.ShapeDtypeStruct((m, nd), jnp.float32),
    )(h, w, b.reshape(1, nd))


def _dinv(rp_lo, rp_hi):
    def body(lo_ref, hi_ref, o_ref):
        d = (hi_ref[...] - lo_ref[...]).astype(jnp.float32)
        o_ref[...] = jnp.where(d > 0, lax.rsqrt(jnp.maximum(d, 1.0)), 0.0)

    return pl.pallas_call(
        body,
        out_shape=jax.ShapeDtypeStruct(rp_lo.shape, jnp.float32),
    )(rp_lo, rp_hi)


@functools.partial(
    pl.kernel,
    out_type=jax.ShapeDtypeStruct((E2P,), jnp.float32),
    mesh=plsc.VectorSubcoreMesh(core_axis_name="c", subcore_axis_name="s"),
    scratch_types=[
        pltpu.VMEM((NPAD,), jnp.float32),
        pltpu.VMEM((EPT,), jnp.int32),
        pltpu.VMEM((EPT,), jnp.int32),
        pltpu.VMEM((EPT,), jnp.float32),
    ],
)
def _norm_sc(dinv_hbm, src_hbm, dst_hbm, norm_hbm, dinv_v, src_v, dst_v, norm_v):
    base = _wid() * EPT
    pltpu.sync_copy(dinv_hbm, dinv_v)
    pltpu.sync_copy(src_hbm.at[pl.ds(base, EPT)], src_v)
    pltpu.sync_copy(dst_hbm.at[pl.ds(base, EPT)], dst_v)

    @pl.loop(0, EPT // 16)
    def _(i):
        s16 = src_v[pl.ds(i * 16, 16)]
        d16 = dst_v[pl.ds(i * 16, 16)]
        a = plsc.load_gather(dinv_v, [s16])
        bb = plsc.load_gather(dinv_v, [d16])
        norm_v[pl.ds(i * 16, 16)] = a * bb

    pltpu.sync_copy(norm_v, norm_hbm.at[pl.ds(base, EPT)])


def _make_agg(d, fuse):
    nb = 320 if d <= 256 else (160 if d == 512 else 64)
    nblocks = NPT // nb
    ch = d // 16

    @functools.partial(
        pl.kernel,
        out_type=jax.ShapeDtypeStruct((NPAD * d,), jnp.float32),
        mesh=plsc.VectorSubcoreMesh(core_axis_name="c", subcore_axis_name="s"),
        scratch_types=[
            pltpu.VMEM((RPLEN,), jnp.int32),
            pltpu.VMEM((nb * d,), jnp.float32),
            pltpu.VMEM((K,), jnp.int32),
            pltpu.VMEM((K,), jnp.int32),
            pltpu.VMEM((K,), jnp.float32),
            pltpu.VMEM((K, d), jnp.float32),
            pltpu.VMEM((d,), jnp.float32),
            pltpu.SemaphoreType.DMA,
        ],
    )
    def agg(h_hbm, rp_hbm, src_hbm, dst_hbm, norm_hbm, b_hbm, out_hbm,
            rp_v, out_v, src_v, dst_v, norm_v, rows_v, b_v, sem):
        w = _wid()
        n0 = w * NPT
        pltpu.sync_copy(rp_hbm.at[pl.ds(n0, RPLEN)], rp_v)
        if fuse:
            pltpu.sync_copy(b_hbm, b_v)

        for blk in range(nblocks):
            bn0 = blk * nb
            node0 = n0 + bn0

            @pl.loop(0, nb * ch)
            def _(i):
                out_v[pl.ds(i * 16, 16)] = jnp.zeros((16,), jnp.float32)

            e0 = rp_v[bn0]
            e1 = rp_v[bn0 + nb]
            cb0 = e0 & (-8)
            nchunk = (e1 - cb0 + (K - 1)) >> 5

            @pl.loop(0, nchunk)
            def _(m):
                cb = cb0 + m * K
                pltpu.sync_copy(src_hbm.at[pl.ds(cb, K)], src_v)
                pltpu.sync_copy(dst_hbm.at[pl.ds(cb, K)], dst_v)
                pltpu.sync_copy(norm_hbm.at[pl.ds(cb, K)], norm_v)
                pltpu.async_copy(h_hbm.at[src_v], rows_v, sem).wait()
                j0 = jnp.maximum(e0 - cb, 0)
                j1 = jnp.minimum(e1 - cb, K)

                @pl.loop(j0, j1)
                def _(j):
                    dl = dst_v[j] - node0
                    nr = norm_v[j]
                    for c in range(ch):
                        plsc.addupdate(
                            out_v.at[pl.ds(dl * d + c * 16, 16)],
                            nr * rows_v[j, pl.ds(c * 16, 16)],
                        )

            if fuse:
                @pl.loop(0, nb)
                def _(r):
                    for c in range(ch):
                        sl = pl.ds(r * d + c * 16, 16)
                        out_v[sl] = jnp.maximum(
                            out_v[sl] + b_v[pl.ds(c * 16, 16)], 0.0)

            pltpu.sync_copy(
                out_v, out_hbm.at[pl.ds(node0 * d, nb * d)])

    return agg


_AGGS = {}


def _agg(h, rowptr, srcs, dsts, norm, b, fuse):
    d = h.shape[1]
    key = (d, fuse)
    if key not in _AGGS:
        _AGGS[key] = _make_agg(d, fuse)
    out = _AGGS[key](h, rowptr, srcs, dsts, norm, b)
    return out.reshape(NPAD, d)


def kernel(x, edge_index, W1, b1, W2, b2, W3, b3, W4, b4, W5, b5, W6, b6,
           W7, b7, W8, b8):
    loop = jnp.arange(N, dtype=edge_index.dtype)
    src = jnp.concatenate([edge_index[0], loop])
    dst = jnp.concatenate([edge_index[1], loop])
    dsts, srcs = lax.sort((dst, src), num_keys=1)
    rowptr = jnp.searchsorted(
        dsts, jnp.arange(NPAD + 8, dtype=jnp.int32)).astype(jnp.int32)
    pad = E2P - E2
    srcs_p = jnp.concatenate(
        [srcs, jnp.zeros((pad,), jnp.int32)]).astype(jnp.int32)
    dsts_p = jnp.concatenate(
        [dsts, jnp.full((pad,), N - 1, jnp.int32)]).astype(jnp.int32)

    rp_lo = rowptr[:NPAD].reshape(80, 128)
    rp_hi = rowptr[1:NPAD + 1].reshape(80, 128)
    dinv = _dinv(rp_lo, rp_hi).reshape(NPAD)
    norm = _norm_sc(dinv, srcs_p, dsts_p)

    x_p = jnp.pad(x, ((0, NPAD - N), (0, 0)))

    ax = _agg(x_p, rowptr, srcs_p, dsts_p, norm, b1, fuse=False)
    h = _mm(ax, W1, b1, relu=True)
    for (w, b) in ((W2, b2), (W3, b3), (W4, b4), (W5, b5), (W6, b6), (W7, b7)):
        t = _mm(h, w, jnp.zeros((w.shape[1],), jnp.float32), relu=False)
        h = _agg(t, rowptr, srcs_p, dsts_p, norm, b, fuse=True)
    a8 = _agg(h, rowptr, srcs_p, dsts_p, norm, b8, fuse=False)
    out = _mm(a8, W8, b8, relu=False)
    return out[:N]


# speedup vs baseline: 2.5553x; 2.5553x over previous
import functools

import jax
import jax.numpy as jnp
from jax import lax
from jax.experimental import pallas as pl
from jax.experimental.pallas import tpu as pltpu
from jax.experimental.pallas import tpu_sc as plsc

N = 10000
E = 320000
E2 = E + N
NW = 32
NPT = 320
NPAD = NW * NPT
EPT = ((E2 + NW * 16 - 1) // (NW * 16)) * 16
E2P = EPT * NW
K = 32
RPLEN = NPT + 16
K16 = K + 16


def _wid():
    return lax.axis_index("s") * 2 + lax.axis_index("c")


def _mm(h, w, b, relu):
    m, kd = h.shape
    nd = w.shape[1]
    bm = 512

    def body(h_ref, w_ref, b_ref, o_ref):
        acc = jnp.dot(h_ref[...], w_ref[...], preferred_element_type=jnp.float32)
        acc = acc + b_ref[...]
        if relu:
            acc = jnp.maximum(acc, 0.0)
        o_ref[...] = acc

    return pl.pallas_call(
        body,
        grid=(m // bm,),
        in_specs=[
            pl.BlockSpec((bm, kd), lambda i: (i, 0)),
            pl.BlockSpec((kd, nd), lambda i: (0, 0)),
            pl.BlockSpec((1, nd), lambda i: (0, 0)),
        ],
        out_specs=pl.BlockSpec((bm, nd), lambda i: (i, 0)),
        out_shape=jax.ShapeDtypeStruct((m, nd), jnp.float32),
    )(h, w, b.reshape(1, nd))


def _dinv(rp_lo, rp_hi):
    def body(lo_ref, hi_ref, o_ref):
        d = (hi_ref[...] - lo_ref[...]).astype(jnp.float32)
        o_ref[...] = jnp.where(d > 0, lax.rsqrt(jnp.maximum(d, 1.0)), 0.0)

    return pl.pallas_call(
        body,
        out_shape=jax.ShapeDtypeStruct(rp_lo.shape, jnp.float32),
    )(rp_lo, rp_hi)


@functools.partial(
    pl.kernel,
    out_type=jax.ShapeDtypeStruct((E2P,), jnp.float32),
    mesh=plsc.VectorSubcoreMesh(core_axis_name="c", subcore_axis_name="s"),
    compiler_params=pltpu.CompilerParams(needs_layout_passes=False),
    scratch_types=[
        pltpu.VMEM((NPAD,), jnp.float32),
        pltpu.VMEM((EPT,), jnp.int32),
        pltpu.VMEM((EPT,), jnp.int32),
        pltpu.VMEM((EPT,), jnp.float32),
    ],
)
def _norm_sc(dinv_hbm, src_hbm, dst_hbm, norm_hbm, dinv_v, src_v, dst_v, norm_v):
    base = _wid() * EPT
    pltpu.sync_copy(dinv_hbm, dinv_v)
    pltpu.sync_copy(src_hbm.at[pl.ds(base, EPT)], src_v)
    pltpu.sync_copy(dst_hbm.at[pl.ds(base, EPT)], dst_v)

    @pl.loop(0, EPT // 16)
    def _(i):
        s16 = src_v[pl.ds(i * 16, 16)]
        d16 = dst_v[pl.ds(i * 16, 16)]
        a = plsc.load_gather(dinv_v, [s16])
        bb = plsc.load_gather(dinv_v, [d16])
        norm_v[pl.ds(i * 16, 16)] = a * bb

    pltpu.sync_copy(norm_v, norm_hbm.at[pl.ds(base, EPT)])


def _make_agg(d, fuse):
    nb = 320 if d <= 256 else (160 if d == 512 else 64)
    nblocks = NPT // nb
    ch = d // 16

    @functools.partial(
        pl.kernel,
        out_type=jax.ShapeDtypeStruct((NPAD * d,), jnp.float32),
        mesh=plsc.VectorSubcoreMesh(core_axis_name="c", subcore_axis_name="s"),
        compiler_params=pltpu.CompilerParams(use_tc_tiling_on_sc=False),
        scratch_types=[
            pltpu.VMEM((RPLEN,), jnp.int32),
            pltpu.VMEM((nb * d,), jnp.float32),
            pltpu.VMEM((K,), jnp.int32),
            pltpu.VMEM((K16,), jnp.int32),
            pltpu.VMEM((K16,), jnp.float32),
            pltpu.VMEM((K, d), jnp.float32),
            pltpu.VMEM((d,), jnp.float32),
            pltpu.SemaphoreType.DMA,
        ],
    )
    def agg(h_hbm, rp_hbm, src_hbm, dst_hbm, norm_hbm, b_hbm, out_hbm,
            rp_v, out_v, src_v, dst_v, norm_v, rows_v, b_v, sem):
        w = _wid()
        n0 = pl.multiple_of(w * NPT, 8)
        pltpu.sync_copy(rp_hbm.at[pl.ds(n0, RPLEN)], rp_v)
        if fuse:
            pltpu.sync_copy(b_hbm, b_v)

        for blk in range(nblocks):
            bn0 = blk * nb
            node0 = n0 + bn0

            @pl.loop(0, nb * ch)
            def _(i):
                out_v[pl.ds(i * 16, 16)] = jnp.zeros((16,), jnp.float32)

            e0 = rp_v[pl.ds(bn0, 16)][0]
            e1 = rp_v[pl.ds(bn0 + nb, 16)][0]
            cb0 = e0 & (-8)
            nchunk = (e1 - cb0 + (K - 1)) >> 5

            @pl.loop(0, nchunk)
            def _(m):
                cb = pl.multiple_of(cb0 + m * K, 8)
                pltpu.sync_copy(src_hbm.at[pl.ds(cb, K)], src_v)
                pltpu.sync_copy(dst_hbm.at[pl.ds(cb, K)], dst_v.at[pl.ds(0, K)])
                pltpu.sync_copy(norm_hbm.at[pl.ds(cb, K)], norm_v.at[pl.ds(0, K)])
                pltpu.async_copy(h_hbm.at[src_v], rows_v, sem).wait()
                j0 = jnp.maximum(e0 - cb, 0)
                j1 = jnp.minimum(e1 - cb, K)

                @pl.loop(j0, j1)
                def _(j):
                    dl = dst_v[pl.ds(j, 16)][0] - node0
                    nr = norm_v[pl.ds(j, 16)][0]
                    for c in range(ch):
                        plsc.addupdate(
                            out_v.at[pl.ds(dl * d + c * 16, 16)],
                            nr * rows_v[j, pl.ds(c * 16, 16)],
                        )

            if fuse:
                @pl.loop(0, nb)
                def _(r):
                    for c in range(ch):
                        sl = pl.ds(r * d + c * 16, 16)
                        out_v[sl] = jnp.maximum(
                            out_v[sl] + b_v[pl.ds(c * 16, 16)], 0.0)

            pltpu.sync_copy(
                out_v, out_hbm.at[pl.ds(pl.multiple_of(node0 * d, 8), nb * d)])

    return agg


_AGGS = {}


def _agg(h, rowptr, srcs, dsts, norm, b, fuse):
    d = h.shape[1]
    key = (d, fuse)
    if key not in _AGGS:
        _AGGS[key] = _make_agg(d, fuse)
    out = _AGGS[key](h, rowptr, srcs, dsts, norm, b)
    return out.reshape(NPAD, d)


def kernel(x, edge_index, W1, b1, W2, b2, W3, b3, W4, b4, W5, b5, W6, b6,
           W7, b7, W8, b8):
    loop = jnp.arange(N, dtype=edge_index.dtype)
    src = jnp.concatenate([edge_index[0], loop])
    dst = jnp.concatenate([edge_index[1], loop])
    dsts, srcs = lax.sort((dst, src), num_keys=1)
    rowptr = jnp.searchsorted(
        dsts, jnp.arange(NPAD + 16, dtype=jnp.int32)).astype(jnp.int32)
    pad = E2P - E2
    srcs_p = jnp.concatenate(
        [srcs, jnp.zeros((pad,), jnp.int32)]).astype(jnp.int32)
    dsts_p = jnp.concatenate(
        [dsts, jnp.full((pad,), N - 1, jnp.int32)]).astype(jnp.int32)

    rp_lo = rowptr[:NPAD].reshape(80, 128)
    rp_hi = rowptr[1:NPAD + 1].reshape(80, 128)
    dinv = _dinv(rp_lo, rp_hi).reshape(NPAD)
    norm = _norm_sc(dinv, srcs_p, dsts_p)

    x_p = jnp.pad(x, ((0, NPAD - N), (0, 0)))

    ax = _agg(x_p, rowptr, srcs_p, dsts_p, norm, b1, fuse=False)
    h = _mm(ax, W1, b1, relu=True)
    for (w, b) in ((W2, b2), (W3, b3), (W4, b4), (W5, b5), (W6, b6), (W7, b7)):
        t = _mm(h, w, jnp.zeros((w.shape[1],), jnp.float32), relu=False)
        h = _agg(t, rowptr, srcs_p, dsts_p, norm, b, fuse=True)
    a8 = _agg(h, rowptr, srcs_p, dsts_p, norm, b8, fuse=False)
    out = _mm(a8, W8, b8, relu=False)
    return out[:N]


# speedup vs baseline: 3.8373x; 1.5017x over previous
import functools

import jax
import jax.numpy as jnp
from jax import lax
from jax.experimental import pallas as pl
from jax.experimental.pallas import tpu as pltpu
from jax.experimental.pallas import tpu_sc as plsc

N = 10000
E = 320000
E2 = E + N
NW = 32
NPT = 320
NPAD = NW * NPT
EPT = ((E2 + NW * 16 - 1) // (NW * 16)) * 16
E2P = EPT * NW
K = 32
RPLEN = NPT + 16
K16 = K + 16


def _wid():
    return lax.axis_index("s") * 2 + lax.axis_index("c")


def _mm(h, w, b, relu):
    m, kd = h.shape
    nd = w.shape[1]
    bm = 512

    def body(h_ref, w_ref, b_ref, o_ref):
        acc = jnp.dot(h_ref[...], w_ref[...], preferred_element_type=jnp.float32)
        acc = acc + b_ref[...]
        if relu:
            acc = jnp.maximum(acc, 0.0)
        o_ref[...] = acc

    return pl.pallas_call(
        body,
        grid=(m // bm,),
        in_specs=[
            pl.BlockSpec((bm, kd), lambda i: (i, 0)),
            pl.BlockSpec((kd, nd), lambda i: (0, 0)),
            pl.BlockSpec((1, nd), lambda i: (0, 0)),
        ],
        out_specs=pl.BlockSpec((bm, nd), lambda i: (i, 0)),
        out_shape=jax.ShapeDtypeStruct((m, nd), jnp.float32),
    )(h, w, b.reshape(1, nd))


def _dinv(rp_lo, rp_hi):
    def body(lo_ref, hi_ref, o_ref):
        d = (hi_ref[...] - lo_ref[...]).astype(jnp.float32)
        o_ref[...] = jnp.where(d > 0, lax.rsqrt(jnp.maximum(d, 1.0)), 0.0)

    return pl.pallas_call(
        body,
        out_shape=jax.ShapeDtypeStruct(rp_lo.shape, jnp.float32),
    )(rp_lo, rp_hi)


@functools.partial(
    pl.kernel,
    out_type=jax.ShapeDtypeStruct((E2P,), jnp.float32),
    mesh=plsc.VectorSubcoreMesh(core_axis_name="c", subcore_axis_name="s"),
    compiler_params=pltpu.CompilerParams(needs_layout_passes=False),
    scratch_types=[
        pltpu.VMEM((NPAD,), jnp.float32),
        pltpu.VMEM((EPT,), jnp.int32),
        pltpu.VMEM((EPT,), jnp.int32),
        pltpu.VMEM((EPT,), jnp.float32),
    ],
)
def _norm_sc(dinv_hbm, src_hbm, dst_hbm, norm_hbm, dinv_v, src_v, dst_v, norm_v):
    base = _wid() * EPT
    pltpu.sync_copy(dinv_hbm, dinv_v)
    pltpu.sync_copy(src_hbm.at[pl.ds(base, EPT)], src_v)
    pltpu.sync_copy(dst_hbm.at[pl.ds(base, EPT)], dst_v)

    @pl.loop(0, EPT // 16)
    def _(i):
        s16 = src_v[pl.ds(i * 16, 16)]
        d16 = dst_v[pl.ds(i * 16, 16)]
        a = plsc.load_gather(dinv_v, [s16])
        bb = plsc.load_gather(dinv_v, [d16])
        norm_v[pl.ds(i * 16, 16)] = a * bb

    pltpu.sync_copy(norm_v, norm_hbm.at[pl.ds(base, EPT)])


SCE = 512


def _make_agg(d, fuse):
    if d <= 128:
        kk, nb = 128, 320
    elif d == 256:
        kk, nb = 128, 160
    elif d == 512:
        kk, nb = 64, 80
    else:
        kk, nb = 32, 32
    nblocks = NPT // nb
    ch = d // 16
    scq = SCE // kk
    kshift = kk.bit_length() - 1

    @functools.partial(
        pl.kernel,
        out_type=jax.ShapeDtypeStruct((NPAD * d,), jnp.float32),
        mesh=plsc.VectorSubcoreMesh(core_axis_name="c", subcore_axis_name="s"),
        compiler_params=pltpu.CompilerParams(use_tc_tiling_on_sc=False),
        scratch_types=[
            pltpu.VMEM((RPLEN,), jnp.int32),
            pltpu.VMEM((nb * d,), jnp.float32),
            pltpu.VMEM((SCE,), jnp.int32),
            pltpu.VMEM((SCE + 16,), jnp.int32),
            pltpu.VMEM((SCE + 16,), jnp.float32),
            pltpu.VMEM((2, kk, d), jnp.float32),
            pltpu.VMEM((d,), jnp.float32),
            pltpu.SemaphoreType.DMA,
            pltpu.SemaphoreType.DMA,
        ],
    )
    def agg(h_hbm, rp_hbm, src_hbm, dst_hbm, norm_hbm, b_hbm, out_hbm,
            rp_v, out_v, ssrc_v, sdst_v, snorm_v, rows_v, b_v, sem0, sem1):
        w = _wid()
        n0 = pl.multiple_of(w * NPT, 8)
        pltpu.sync_copy(rp_hbm.at[pl.ds(n0, RPLEN)], rp_v)
        if fuse:
            pltpu.sync_copy(b_hbm, b_v)
        sems = (sem0, sem1)

        @pl.loop(0, nblocks)
        def _(blk):
            bn0 = blk * nb
            node0 = n0 + bn0

            @pl.loop(0, nb * ch)
            def _(i):
                out_v[pl.ds(i * 16, 16)] = jnp.zeros((16,), jnp.float32)

            e0 = rp_v[pl.ds(bn0, 16)][0]
            e1 = rp_v[pl.ds(bn0 + nb, 16)][0]
            cb0 = e0 & (-8)
            nsc = (e1 - cb0 + (SCE - 1)) >> 9

            @pl.loop(0, nsc)
            def _(s):
                sb = pl.multiple_of(cb0 + s * SCE, 8)
                pltpu.sync_copy(src_hbm.at[pl.ds(sb, SCE)], ssrc_v)
                pltpu.sync_copy(dst_hbm.at[pl.ds(sb, SCE)],
                                sdst_v.at[pl.ds(0, SCE)])
                pltpu.sync_copy(norm_hbm.at[pl.ds(sb, SCE)],
                                snorm_v.at[pl.ds(0, SCE)])
                q1 = jnp.minimum((e1 - sb + (kk - 1)) >> kshift, scq)

                def fire(q, par):
                    pltpu.async_copy(
                        h_hbm.at[ssrc_v.at[pl.ds(q * kk, kk)]],
                        rows_v.at[par], sems[par])

                def drain(par):
                    pltpu.make_async_copy(
                        h_hbm.at[ssrc_v.at[pl.ds(0, kk)]],
                        rows_v.at[par], sems[par]).wait()

                def process(q, par):
                    j0 = jnp.maximum(e0 - sb - q * kk, 0)
                    j1 = jnp.minimum(e1 - sb - q * kk, kk)

                    @pl.loop(j0, j1)
                    def _(j):
                        jj = q * kk + j
                        dl = sdst_v[pl.ds(jj, 16)][0] - node0
                        nr = snorm_v[pl.ds(jj, 16)][0]
                        for c in range(ch):
                            plsc.addupdate(
                                out_v.at[pl.ds(dl * d + c * 16, 16)],
                                nr * rows_v.at[par][j, pl.ds(c * 16, 16)],
                            )

                @pl.when(q1 > 0)
                def _():
                    fire(0, 0)

                @pl.when(q1 > 1)
                def _():
                    fire(1, 1)

                @pl.loop(0, (q1 + 1) >> 1)
                def _(h2):
                    q = h2 * 2
                    drain(0)
                    process(q, 0)

                    @pl.when(q + 2 < q1)
                    def _():
                        fire(q + 2, 0)

                    @pl.when(q + 1 < q1)
                    def _():
                        drain(1)
                        process(q + 1, 1)

                        @pl.when(q + 3 < q1)
                        def _():
                            fire(q + 3, 1)

            if fuse:
                @pl.loop(0, nb)
                def _(r):
                    for c in range(ch):
                        sl = pl.ds(r * d + c * 16, 16)
                        out_v[sl] = jnp.maximum(
                            out_v[sl] + b_v[pl.ds(c * 16, 16)], 0.0)

            pltpu.sync_copy(
                out_v, out_hbm.at[pl.ds(pl.multiple_of(node0 * d, 8), nb * d)])

    return agg


_AGGS = {}


def _agg(h, rowptr, srcs, dsts, norm, b, fuse):
    d = h.shape[1]
    key = (d, fuse)
    if key not in _AGGS:
        _AGGS[key] = _make_agg(d, fuse)
    out = _AGGS[key](h, rowptr, srcs, dsts, norm, b)
    return out.reshape(NPAD, d)


def kernel(x, edge_index, W1, b1, W2, b2, W3, b3, W4, b4, W5, b5, W6, b6,
           W7, b7, W8, b8):
    loop = jnp.arange(N, dtype=edge_index.dtype)
    src = jnp.concatenate([edge_index[0], loop])
    dst = jnp.concatenate([edge_index[1], loop])
    dsts, srcs = lax.sort((dst, src), num_keys=1)
    rowptr = jnp.searchsorted(
        dsts, jnp.arange(NPAD + 16, dtype=jnp.int32)).astype(jnp.int32)
    pad = E2P - E2
    srcs_p = jnp.concatenate(
        [srcs, jnp.zeros((pad,), jnp.int32)]).astype(jnp.int32)
    dsts_p = jnp.concatenate(
        [dsts, jnp.full((pad,), N - 1, jnp.int32)]).astype(jnp.int32)

    rp_lo = rowptr[:NPAD].reshape(80, 128)
    rp_hi = rowptr[1:NPAD + 1].reshape(80, 128)
    dinv = _dinv(rp_lo, rp_hi).reshape(NPAD)
    norm = _norm_sc(dinv, srcs_p, dsts_p)

    x_p = jnp.pad(x, ((0, NPAD - N), (0, 0)))

    ax = _agg(x_p, rowptr, srcs_p, dsts_p, norm, b1, fuse=False)
    h = _mm(ax, W1, b1, relu=True)
    for (w, b) in ((W2, b2), (W3, b3), (W4, b4), (W5, b5), (W6, b6), (W7, b7)):
        t = _mm(h, w, jnp.zeros((w.shape[1],), jnp.float32), relu=False)
        h = _agg(t, rowptr, srcs_p, dsts_p, norm, b, fuse=True)
    a8 = _agg(h, rowptr, srcs_p, dsts_p, norm, b8, fuse=False)
    out = _mm(a8, W8, b8, relu=False)
    return out[:N]
